# baseline (speedup 1.0000x reference)
import functools

import jax
import jax.numpy as jnp
from jax import lax
from jax.experimental import pallas as pl
from jax.experimental.pallas import tpu as pltpu
from jax.experimental.pallas import tpu_sc as plsc

D = 128
D3 = 3 * D
NKEY = 10000
NCTR = 10000
NP = 10240
NE = 320000
CHUNK = 128
NTILES = 32
CPT = 80
EP = NTILES * CPT * CHUNK
NACC = 10112
RPSA = NACC // 16
PAD_IDX = 10000
RPT = NP // NTILES
RPS = NP // 16
NBLK = NP // 128
KEY_BUCKETS = 100000
CTR_BUCKETS = 100000

_PREC = jax.lax.Precision.HIGHEST

_MESH = dict(core_axis_name="c", subcore_axis_name="s", num_cores=2, num_subcores=16)


def _sc_prologue_body(kemb, cemb, kh, ch,
                      hkey_out, crow_out,
                      idx128, idx64, rows128, rows64, sem):
    cid = lax.axis_index("c")
    sid = lax.axis_index("s")
    wid = sid * 2 + cid
    gb = wid * RPT
    for off, cnt, ib, rb in ((0, CHUNK, idx128, rows128),
                             (CHUNK, CHUNK, idx128, rows128),
                             (2 * CHUNK, 64, idx64, rows64)):
        pltpu.sync_copy(kh.at[pl.ds(gb + off, cnt)], ib)
        pltpu.async_copy(kemb.at[ib], rb, sem).wait()
        pltpu.sync_copy(rb, hkey_out.at[pl.ds(gb + off, cnt)])
        pltpu.sync_copy(ch.at[pl.ds(gb + off, cnt)], ib)
        pltpu.async_copy(cemb.at[ib], rb, sem).wait()
        pltpu.sync_copy(rb, crow_out.at[pl.ds(gb + off, cnt)])


@functools.cache
def _sc_prologue():
    return pl.kernel(
        _sc_prologue_body,
        mesh=plsc.VectorSubcoreMesh(**_MESH),
        out_type=[
            jax.ShapeDtypeStruct((NP, D), jnp.float32),
            jax.ShapeDtypeStruct((NP, D), jnp.float32),
        ],
        scratch_types=[
            pltpu.VMEM((CHUNK,), jnp.int32),
            pltpu.VMEM((64,), jnp.int32),
            pltpu.VMEM((CHUNK, D), jnp.float32),
            pltpu.VMEM((64, D), jnp.float32),
            pltpu.SemaphoreType.DMA,
        ],
    )


def _sc_agg_body(table, gidx2, sidx2, zrows, out,
                 gidx_v, sidx_v, rows_v, acc, gsem):
    cid = lax.axis_index("c")
    sid = lax.axis_index("s")
    wid = sid * 2 + cid
    slab = pl.ds(sid * RPSA, RPSA)

    pltpu.sync_copy(zrows, acc.at[slab])
    pltpu.sync_copy(gidx2.at[pl.ds(wid * CPT, CPT)], gidx_v)
    pltpu.sync_copy(sidx2.at[pl.ds(wid * CPT, CPT)], sidx_v)
    plsc.subcore_barrier()

    def body(t, carry):
        pltpu.async_copy(table.at[gidx_v.at[t]], rows_v, gsem).wait()
        pltpu.sync_copy(rows_v, acc.at[sidx_v.at[t]], add=True)
        return carry

    lax.fori_loop(jnp.int32(0), jnp.int32(CPT), body, jnp.int32(0))
    plsc.subcore_barrier()
    pltpu.sync_copy(acc.at[slab], out.at[cid, slab])


@functools.cache
def _sc_agg():
    return pl.kernel(
        _sc_agg_body,
        mesh=plsc.VectorSubcoreMesh(**_MESH),
        out_type=jax.ShapeDtypeStruct((2, NP, D), jnp.float32),
        scratch_types=[
            pltpu.VMEM((CPT, CHUNK), jnp.int32),
            pltpu.VMEM((CPT, CHUNK), jnp.int32),
            pltpu.VMEM((CHUNK, D), jnp.float32),
            pltpu.VMEM_SHARED((NACC, D), jnp.float32),
            pltpu.SemaphoreType.DMA,
        ],
    )


def _sc_deg_body(src2, dst2, ones_hbm, zrows, outk, outc,
                 idx_v, ones_v, acc):
    cid = lax.axis_index("c")
    sid = lax.axis_index("s")
    wid = sid * 2 + cid
    slab = pl.ds(sid * RPS, RPS)
    pltpu.sync_copy(ones_hbm, ones_v)

    def epoch(eidx2, outref):
        pltpu.sync_copy(zrows, acc.at[slab])
        pltpu.sync_copy(eidx2.at[pl.ds(wid * CPT, CPT)], idx_v)
        plsc.subcore_barrier()

        def body(t, carry):
            pltpu.sync_copy(ones_v, acc.at[idx_v.at[t]], add=True)
            return carry

        lax.fori_loop(jnp.int32(0), jnp.int32(CPT), body, jnp.int32(0))
        plsc.subcore_barrier()
        pltpu.sync_copy(acc.at[slab], outref.at[cid, slab])

    epoch(src2, outk)
    epoch(dst2, outc)


@functools.cache
def _sc_deg():
    return pl.kernel(
        _sc_deg_body,
        mesh=plsc.VectorSubcoreMesh(**_MESH),
        out_type=[
            jax.ShapeDtypeStruct((2, NP, D), jnp.float32),
            jax.ShapeDtypeStruct((2, NP, D), jnp.float32),
        ],
        scratch_types=[
            pltpu.VMEM((CPT, CHUNK), jnp.int32),
            pltpu.VMEM((CHUNK, D), jnp.float32),
            pltpu.VMEM_SHARED((NP, D), jnp.float32),
        ],
    )


def _full(shape):
    return pl.BlockSpec(shape, lambda i: tuple(jnp.int32(0) for _ in shape))


def _rows(width):
    return pl.BlockSpec((128, width), lambda i: (i, jnp.int32(0)))


def _mlp_blk(x, w1t, b1, w2t, b2):
    h = jnp.maximum(jnp.dot(x, w1t, precision=_PREC) + b1, 0.0)
    return jnp.dot(h, w2t, precision=_PREC) + b2


def _tab_mask():
    i = pl.program_id(0)
    rid = i * 128 + lax.broadcasted_iota(jnp.int32, (128, 1), 0)
    return rid < NKEY


def _tc_prologue_body(crow_ref, y_ref, vwt_ref, vb_ref,
                      w1t_ref, b1_ref, w2t_ref, b2_ref,
                      hctr_ref, tab_ref):
    y = y_ref[...]
    l1p = jnp.log1p(jnp.maximum(y, 0.0))
    pre = y * vwt_ref[0:1, :] + l1p * vwt_ref[1:2, :] + vb_ref[...]
    hc = crow_ref[...] + jnp.maximum(pre, 0.0)
    hctr_ref[...] = hc
    tab = _mlp_blk(hc, w1t_ref[...], b1_ref[...], w2t_ref[...], b2_ref[...])
    tab_ref[...] = jnp.where(_tab_mask(), tab, 0.0)


_tc_prologue = pl.pallas_call(
    _tc_prologue_body,
    grid=(NBLK,),
    in_specs=[_rows(D), _rows(1), _full((2, D)), _full((1, D)),
              _full((D, D)), _full((1, D)), _full((D, D)), _full((1, D))],
    out_specs=[_rows(D), _rows(D)],
    out_shape=[jax.ShapeDtypeStruct((NP, D), jnp.float32),
               jax.ShapeDtypeStruct((NP, D), jnp.float32)],
)


def _mk_update(with_table):
    def body(p_ref, dp_ref, h_ref, wiht_ref, bih_ref, whht_ref, bhh_ref,
             g_ref, b_ref, *rest):
        x = (p_ref[0] + p_ref[1]) / jnp.maximum(dp_ref[0, :, 0:1] + dp_ref[1, :, 0:1], 1.0)
        h = h_ref[...]
        gi = jnp.dot(x, wiht_ref[...], precision=_PREC) + bih_ref[...]
        gh = jnp.dot(h, whht_ref[...], precision=_PREC) + bhh_ref[...]
        r = jax.nn.sigmoid(gi[:, :D] + gh[:, :D])
        z = jax.nn.sigmoid(gi[:, D:2 * D] + gh[:, D:2 * D])
        n = jnp.tanh(gi[:, 2 * D:] + r * gh[:, 2 * D:])
        hn = (1.0 - z) * n + z * h
        mu = jnp.mean(hn, axis=1, keepdims=True)
        var = jnp.mean((hn - mu) ** 2, axis=1, keepdims=True)
        ln = (hn - mu) / jnp.sqrt(var + 1e-5) * g_ref[...] + b_ref[...]
        if with_table:
            w1t_ref, b1_ref, w2t_ref, b2_ref, hout_ref, tab_ref = rest
            hout_ref[...] = ln
            tab = _mlp_blk(ln, w1t_ref[...], b1_ref[...], w2t_ref[...], b2_ref[...])
            tab_ref[...] = jnp.where(_tab_mask(), tab, 0.0)
        else:
            (hout_ref,) = rest
            hout_ref[...] = ln

    in_specs = [
        pl.BlockSpec((2, 128, D), lambda i: (jnp.int32(0), i, jnp.int32(0))),
        pl.BlockSpec((2, 128, D), lambda i: (jnp.int32(0), i, jnp.int32(0))),
        _rows(D),
        _full((D, D3)), _full((1, D3)), _full((D, D3)), _full((1, D3)),
        _full((1, D)), _full((1, D)),
    ]
    out_shape = [jax.ShapeDtypeStruct((NP, D), jnp.float32)]
    out_specs = [_rows(D)]
    if with_table:
        in_specs += [_full((D, D)), _full((1, D)), _full((D, D)), _full((1, D))]
        out_shape += [jax.ShapeDtypeStruct((NP, D), jnp.float32)]
        out_specs += [_rows(D)]
    return pl.pallas_call(
        body, grid=(NBLK,), in_specs=in_specs,
        out_specs=out_specs, out_shape=out_shape,
    )


_tc_update_tab = _mk_update(True)
_tc_update_notab = _mk_update(False)


def _tc_epilogue_body(h_ref, cw1t, cb1, cw2t, cb2, vw1t, vb1, vw2t, vb2,
                      hkc_ref, hkv_ref, sc_ref, sv_ref):
    i = pl.program_id(0)
    h = h_ref[...]
    hc = _mlp_blk(h, cw1t[...], cb1[...], cw2t[...], cb2[...])
    hv = _mlp_blk(h, vw1t[...], vb1[...], vw2t[...], vb2[...])
    hkc_ref[...] = hc
    hkv_ref[...] = hv
    rows = i * 128 + lax.broadcasted_iota(jnp.int32, (128, 1), 0)
    mask = rows < NKEY

    @pl.when(i == 0)
    def _():
        sc_ref[...] = jnp.zeros((1, D), jnp.float32)
        sv_ref[...] = jnp.zeros((1, D), jnp.float32)

    sc_ref[...] += jnp.sum(jnp.where(mask, hc, 0.0), axis=0, keepdims=True)
    sv_ref[...] += jnp.sum(jnp.where(mask, hv, 0.0), axis=0, keepdims=True)


_tc_epilogue = pl.pallas_call(
    _tc_epilogue_body,
    grid=(NBLK,),
    in_specs=[_rows(D)] + [_full((D, D)), _full((1, D)), _full((D, D)), _full((1, D))] * 2,
    out_specs=[_rows(D), _rows(D),
               pl.BlockSpec((1, D), lambda i: (jnp.int32(0), jnp.int32(0))),
               pl.BlockSpec((1, D), lambda i: (jnp.int32(0), jnp.int32(0)))],
    out_shape=[jax.ShapeDtypeStruct((NP, D), jnp.float32),
               jax.ShapeDtypeStruct((NP, D), jnp.float32),
               jax.ShapeDtypeStruct((1, D), jnp.float32),
               jax.ShapeDtypeStruct((1, D), jnp.float32)],
)


def _tc_pool_body(sc_ref, sv_ref, cw1t, cb1, cw2t, cb2, vw1t, vb1, vw2t, vb2,
                  zc_ref, zv_ref):
    zc_ref[...] = _mlp_blk(sc_ref[...] * (1.0 / NKEY), cw1t[...], cb1[...], cw2t[...], cb2[...])
    zv_ref[...] = _mlp_blk(sv_ref[...] * (1.0 / NKEY), vw1t[...], vb1[...], vw2t[...], vb2[...])


_tc_pool = pl.pallas_call(
    _tc_pool_body,
    out_shape=[jax.ShapeDtypeStruct((1, D), jnp.float32),
               jax.ShapeDtypeStruct((1, D), jnp.float32)],
)


def _mlp_args(p):
    return (p["w1"].T.astype(jnp.float32), p["b1"].astype(jnp.float32).reshape(1, D),
            p["w2"].T.astype(jnp.float32), p["b2"].astype(jnp.float32).reshape(1, D))


def kernel(y, edge_index, key_ids, counter_ids_global, params):
    p = params
    f32 = jnp.float32
    mult = jnp.uint64(2654435761)
    kh = ((key_ids.astype(jnp.uint64) * mult) % jnp.uint64(KEY_BUCKETS)).astype(jnp.int32)
    ch = ((counter_ids_global.astype(jnp.uint64) * mult) % jnp.uint64(CTR_BUCKETS)).astype(jnp.int32)
    khp = jnp.concatenate([kh, jnp.zeros((NP - NKEY,), jnp.int32)])
    chp = jnp.concatenate([ch, jnp.zeros((NP - NCTR,), jnp.int32)])
    padI = jnp.full((EP - NE,), PAD_IDX, jnp.int32)
    src = jnp.concatenate([edge_index[0].astype(jnp.int32), padI]).reshape(NTILES * CPT, CHUNK)
    dst = jnp.concatenate([edge_index[1].astype(jnp.int32), padI]).reshape(NTILES * CPT, CHUNK)
    zrows = jnp.zeros((RPS, D), f32)
    zrows_a = jnp.zeros((RPSA, D), f32)
    yp = jnp.concatenate([y.astype(f32), jnp.zeros((NP - NCTR,), f32)]).reshape(NP, 1)

    kemb = p["key_emb"].astype(f32)
    cemb = p["ctr_emb"].astype(f32)
    vwt = p["val_w"].T.astype(f32)
    vb = p["val_b"].astype(f32).reshape(1, D)
    c2k = _mlp_args(p["msg_c2k"])
    k2c = _mlp_args(p["msg_k2c"])

    def gru_args(g):
        return (g["wih"].T.astype(f32), g["bih"].astype(f32).reshape(1, D3),
                g["whh"].T.astype(f32), g["bhh"].astype(f32).reshape(1, D3))

    gk = gru_args(p["gru_key"])
    gc = gru_args(p["gru_ctr"])
    lnk = (p["ln_key_g"].astype(f32).reshape(1, D), p["ln_key_b"].astype(f32).reshape(1, D))
    lnc = (p["ln_ctr_g"].astype(f32).reshape(1, D), p["ln_ctr_b"].astype(f32).reshape(1, D))

    hkey, crows = _sc_prologue()(kemb, cemb, khp, chp)
    ones128 = jnp.ones((CHUNK, D), f32)
    degk, degc = _sc_deg()(src, dst, ones128, zrows)
    hctr, tab = _tc_prologue(crows, yp, vwt, vb, *c2k)
    agg_fn = _sc_agg()
    for l in range(3):
        aggp = agg_fn(tab, dst, src, zrows_a)
        hkey, tab2 = _tc_update_tab(aggp, degk, hkey, *gk, *lnk, *k2c)
        aggp2 = agg_fn(tab2, src, dst, zrows_a)
        if l < 2:
            hctr, tab = _tc_update_tab(aggp2, degc, hctr, *gc, *lnc, *c2k)
        else:
            (hctr,) = _tc_update_notab(aggp2, degc, hctr, *gc, *lnc)
    hkc, hkv, sc_, sv_ = _tc_epilogue(hkey, *_mlp_args(p["proj_key_c"]),
                                      *_mlp_args(p["proj_key_v"]))
    zc, zv = _tc_pool(sc_, sv_, *_mlp_args(p["pool_zc"]), *_mlp_args(p["pool_zv"]))
    f64 = jnp.float64
    return (zc.reshape(D).astype(f64), zv.reshape(D).astype(f64),
            hkc[:NKEY].astype(f64), hkv[:NKEY].astype(f64),
            hctr[:NCTR].astype(f64))


# speedup vs baseline: 1.3787x; 1.3787x over previous
import functools

import jax
import jax.numpy as jnp
from jax import lax
from jax.experimental import pallas as pl
from jax.experimental.pallas import tpu as pltpu
from jax.experimental.pallas import tpu_sc as plsc

D = 128
D3 = 3 * D
NKEY = 10000
NCTR = 10000
NP = 10240
NE = 320000
CHUNK = 128
NTILES = 32
CPT = 79
EP = NTILES * CPT * CHUNK
PAD_IDX = 10000
RPT = NP // NTILES
RPS = NP // 16
NBLK = NP // 128
KEY_BUCKETS = 100000
CTR_BUCKETS = 100000

_PREC = jax.lax.Precision.HIGHEST

_MESH = dict(core_axis_name="c", subcore_axis_name="s", num_cores=2, num_subcores=16)


def _sc_prologue_body(kemb, cemb, kh, ch,
                      hkey_out, crow_out,
                      idx128, idx64, rows128, rows64, sem):
    cid = lax.axis_index("c")
    sid = lax.axis_index("s")
    wid = sid * 2 + cid
    gb = wid * RPT
    for off, cnt, ib, rb in ((0, CHUNK, idx128, rows128),
                             (CHUNK, CHUNK, idx128, rows128),
                             (2 * CHUNK, 64, idx64, rows64)):
        pltpu.sync_copy(kh.at[pl.ds(gb + off, cnt)], ib)
        pltpu.async_copy(kemb.at[ib], rb, sem).wait()
        pltpu.sync_copy(rb, hkey_out.at[pl.ds(gb + off, cnt)])
        pltpu.sync_copy(ch.at[pl.ds(gb + off, cnt)], ib)
        pltpu.async_copy(cemb.at[ib], rb, sem).wait()
        pltpu.sync_copy(rb, crow_out.at[pl.ds(gb + off, cnt)])


@functools.cache
def _sc_prologue():
    return pl.kernel(
        _sc_prologue_body,
        mesh=plsc.VectorSubcoreMesh(**_MESH),
        out_type=[
            jax.ShapeDtypeStruct((NP, D), jnp.float32),
            jax.ShapeDtypeStruct((NP, D), jnp.float32),
        ],
        scratch_types=[
            pltpu.VMEM((CHUNK,), jnp.int32),
            pltpu.VMEM((64,), jnp.int32),
            pltpu.VMEM((CHUNK, D), jnp.float32),
            pltpu.VMEM((64, D), jnp.float32),
            pltpu.SemaphoreType.DMA,
        ],
    )


def _sc_agg_body(table, gidx, sidx, zrows, out, gi_v, si_v, rows_v, acc, sem):
    cid = lax.axis_index("c")
    sid = lax.axis_index("s")
    wid = sid * 2 + cid
    pltpu.sync_copy(zrows, acc.at[pl.ds(sid * RPS, RPS)])
    plsc.subcore_barrier()

    def body(t, carry):
        gbase = (wid * CPT + t) * CHUNK
        pltpu.sync_copy(gidx.at[pl.ds(gbase, CHUNK)], gi_v)
        pltpu.sync_copy(sidx.at[pl.ds(gbase, CHUNK)], si_v)
        pltpu.async_copy(table.at[gi_v], rows_v, sem).wait()
        pltpu.sync_copy(rows_v, acc.at[si_v], add=True)
        return carry

    lax.fori_loop(jnp.int32(0), jnp.int32(CPT), body, jnp.int32(0))
    plsc.subcore_barrier()
    pltpu.sync_copy(acc.at[pl.ds(sid * RPS, RPS)],
                    out.at[cid, pl.ds(sid * RPS, RPS)])


@functools.cache
def _sc_agg():
    return pl.kernel(
        _sc_agg_body,
        mesh=plsc.VectorSubcoreMesh(**_MESH),
        out_type=jax.ShapeDtypeStruct((2, NP, D), jnp.float32),
        scratch_types=[
            pltpu.VMEM((CHUNK,), jnp.int32),
            pltpu.VMEM((CHUNK,), jnp.int32),
            pltpu.VMEM((CHUNK, D), jnp.float32),
            pltpu.VMEM_SHARED((NP, D), jnp.float32),
            pltpu.SemaphoreType.DMA,
        ],
    )


def _sc_deg_body(srci, dsti, ones_hbm, zrows, outk, outc, idx_v, ones_v, acc, sem):
    cid = lax.axis_index("c")
    sid = lax.axis_index("s")
    wid = sid * 2 + cid
    slab = pl.ds(sid * RPS, RPS)
    pltpu.sync_copy(zrows, acc.at[slab])
    pltpu.sync_copy(ones_hbm, ones_v)
    plsc.subcore_barrier()

    def body_src(t, carry):
        gbase = (wid * CPT + t) * CHUNK
        pltpu.sync_copy(srci.at[pl.ds(gbase, CHUNK)], idx_v)
        pltpu.sync_copy(ones_v, acc.at[idx_v], add=True)
        return carry

    lax.fori_loop(jnp.int32(0), jnp.int32(CPT), body_src, jnp.int32(0))
    plsc.subcore_barrier()
    pltpu.sync_copy(acc.at[slab], outk.at[cid, slab])
    pltpu.sync_copy(zrows, acc.at[slab])
    plsc.subcore_barrier()

    def body_dst(t, carry):
        gbase = (wid * CPT + t) * CHUNK
        pltpu.sync_copy(dsti.at[pl.ds(gbase, CHUNK)], idx_v)
        pltpu.sync_copy(ones_v, acc.at[idx_v], add=True)
        return carry

    lax.fori_loop(jnp.int32(0), jnp.int32(CPT), body_dst, jnp.int32(0))
    plsc.subcore_barrier()
    pltpu.sync_copy(acc.at[slab], outc.at[cid, slab])


@functools.cache
def _sc_deg():
    return pl.kernel(
        _sc_deg_body,
        mesh=plsc.VectorSubcoreMesh(**_MESH),
        out_type=[
            jax.ShapeDtypeStruct((2, NP, D), jnp.float32),
            jax.ShapeDtypeStruct((2, NP, D), jnp.float32),
        ],
        scratch_types=[
            pltpu.VMEM((CHUNK,), jnp.int32),
            pltpu.VMEM((CHUNK, D), jnp.float32),
            pltpu.VMEM_SHARED((NP, D), jnp.float32),
            pltpu.SemaphoreType.DMA,
        ],
    )


def _full(shape):
    return pl.BlockSpec(shape, lambda i: tuple(jnp.int32(0) for _ in shape))


def _rows(width):
    return pl.BlockSpec((128, width), lambda i: (i, jnp.int32(0)))


def _mlp_blk(x, w1t, b1, w2t, b2):
    h = jnp.maximum(jnp.dot(x, w1t, precision=_PREC) + b1, 0.0)
    return jnp.dot(h, w2t, precision=_PREC) + b2


def _tc_prologue_body(crow_ref, y_ref, vwt_ref, vb_ref,
                      w1t_ref, b1_ref, w2t_ref, b2_ref,
                      hctr_ref, tab_ref):
    y = y_ref[...]
    l1p = jnp.log1p(jnp.maximum(y, 0.0))
    pre = y * vwt_ref[0:1, :] + l1p * vwt_ref[1:2, :] + vb_ref[...]
    hc = crow_ref[...] + jnp.maximum(pre, 0.0)
    hctr_ref[...] = hc
    tab_ref[...] = _mlp_blk(hc, w1t_ref[...], b1_ref[...], w2t_ref[...], b2_ref[...])


_tc_prologue = pl.pallas_call(
    _tc_prologue_body,
    grid=(NBLK,),
    in_specs=[_rows(D), _rows(1), _full((2, D)), _full((1, D)),
              _full((D, D)), _full((1, D)), _full((D, D)), _full((1, D))],
    out_specs=[_rows(D), _rows(D)],
    out_shape=[jax.ShapeDtypeStruct((NP, D), jnp.float32),
               jax.ShapeDtypeStruct((NP, D), jnp.float32)],
)


def _mk_update(with_table):
    def body(p_ref, dp_ref, h_ref, wiht_ref, bih_ref, whht_ref, bhh_ref,
             g_ref, b_ref, *rest):
        x = (p_ref[0] + p_ref[1]) / jnp.maximum(dp_ref[0, :, 0:1] + dp_ref[1, :, 0:1], 1.0)
        h = h_ref[...]
        gi = jnp.dot(x, wiht_ref[...], precision=_PREC) + bih_ref[...]
        gh = jnp.dot(h, whht_ref[...], precision=_PREC) + bhh_ref[...]
        r = jax.nn.sigmoid(gi[:, :D] + gh[:, :D])
        z = jax.nn.sigmoid(gi[:, D:2 * D] + gh[:, D:2 * D])
        n = jnp.tanh(gi[:, 2 * D:] + r * gh[:, 2 * D:])
        hn = (1.0 - z) * n + z * h
        mu = jnp.mean(hn, axis=1, keepdims=True)
        var = jnp.mean((hn - mu) ** 2, axis=1, keepdims=True)
        ln = (hn - mu) / jnp.sqrt(var + 1e-5) * g_ref[...] + b_ref[...]
        if with_table:
            w1t_ref, b1_ref, w2t_ref, b2_ref, hout_ref, tab_ref = rest
            hout_ref[...] = ln
            tab_ref[...] = _mlp_blk(ln, w1t_ref[...], b1_ref[...], w2t_ref[...], b2_ref[...])
        else:
            (hout_ref,) = rest
            hout_ref[...] = ln

    in_specs = [
        pl.BlockSpec((2, 128, D), lambda i: (jnp.int32(0), i, jnp.int32(0))),
        pl.BlockSpec((2, 128, D), lambda i: (jnp.int32(0), i, jnp.int32(0))),
        _rows(D),
        _full((D, D3)), _full((1, D3)), _full((D, D3)), _full((1, D3)),
        _full((1, D)), _full((1, D)),
    ]
    out_shape = [jax.ShapeDtypeStruct((NP, D), jnp.float32)]
    out_specs = [_rows(D)]
    if with_table:
        in_specs += [_full((D, D)), _full((1, D)), _full((D, D)), _full((1, D))]
        out_shape += [jax.ShapeDtypeStruct((NP, D), jnp.float32)]
        out_specs += [_rows(D)]
    return pl.pallas_call(
        body, grid=(NBLK,), in_specs=in_specs,
        out_specs=out_specs, out_shape=out_shape,
    )


_tc_update_tab = _mk_update(True)
_tc_update_notab = _mk_update(False)


def _tc_epilogue_body(h_ref, cw1t, cb1, cw2t, cb2, vw1t, vb1, vw2t, vb2,
                      hkc_ref, hkv_ref, sc_ref, sv_ref):
    i = pl.program_id(0)
    h = h_ref[...]
    hc = _mlp_blk(h, cw1t[...], cb1[...], cw2t[...], cb2[...])
    hv = _mlp_blk(h, vw1t[...], vb1[...], vw2t[...], vb2[...])
    hkc_ref[...] = hc
    hkv_ref[...] = hv
    rows = i * 128 + lax.broadcasted_iota(jnp.int32, (128, 1), 0)
    mask = rows < NKEY

    @pl.when(i == 0)
    def _():
        sc_ref[...] = jnp.zeros((1, D), jnp.float32)
        sv_ref[...] = jnp.zeros((1, D), jnp.float32)

    sc_ref[...] += jnp.sum(jnp.where(mask, hc, 0.0), axis=0, keepdims=True)
    sv_ref[...] += jnp.sum(jnp.where(mask, hv, 0.0), axis=0, keepdims=True)


_tc_epilogue = pl.pallas_call(
    _tc_epilogue_body,
    grid=(NBLK,),
    in_specs=[_rows(D)] + [_full((D, D)), _full((1, D)), _full((D, D)), _full((1, D))] * 2,
    out_specs=[_rows(D), _rows(D),
               pl.BlockSpec((1, D), lambda i: (jnp.int32(0), jnp.int32(0))),
               pl.BlockSpec((1, D), lambda i: (jnp.int32(0), jnp.int32(0)))],
    out_shape=[jax.ShapeDtypeStruct((NP, D), jnp.float32),
               jax.ShapeDtypeStruct((NP, D), jnp.float32),
               jax.ShapeDtypeStruct((1, D), jnp.float32),
               jax.ShapeDtypeStruct((1, D), jnp.float32)],
)


def _tc_pool_body(sc_ref, sv_ref, cw1t, cb1, cw2t, cb2, vw1t, vb1, vw2t, vb2,
                  zc_ref, zv_ref):
    zc_ref[...] = _mlp_blk(sc_ref[...] * (1.0 / NKEY), cw1t[...], cb1[...], cw2t[...], cb2[...])
    zv_ref[...] = _mlp_blk(sv_ref[...] * (1.0 / NKEY), vw1t[...], vb1[...], vw2t[...], vb2[...])


_tc_pool = pl.pallas_call(
    _tc_pool_body,
    out_shape=[jax.ShapeDtypeStruct((1, D), jnp.float32),
               jax.ShapeDtypeStruct((1, D), jnp.float32)],
)


def _mlp_args(p):
    return (p["w1"].T.astype(jnp.float32), p["b1"].astype(jnp.float32).reshape(1, D),
            p["w2"].T.astype(jnp.float32), p["b2"].astype(jnp.float32).reshape(1, D))


def kernel(y, edge_index, key_ids, counter_ids_global, params):
    p = params
    f32 = jnp.float32
    mult = jnp.uint64(2654435761)
    kh = ((key_ids.astype(jnp.uint64) * mult) % jnp.uint64(KEY_BUCKETS)).astype(jnp.int32)
    ch = ((counter_ids_global.astype(jnp.uint64) * mult) % jnp.uint64(CTR_BUCKETS)).astype(jnp.int32)
    khp = jnp.concatenate([kh, jnp.zeros((NP - NKEY,), jnp.int32)])
    chp = jnp.concatenate([ch, jnp.zeros((NP - NCTR,), jnp.int32)])
    pad = jnp.full((EP - NE,), PAD_IDX, jnp.int32)
    src = jnp.concatenate([edge_index[0].astype(jnp.int32), pad])
    dst = jnp.concatenate([edge_index[1].astype(jnp.int32), pad])
    zrows = jnp.zeros((RPS, D), f32)
    yp = jnp.concatenate([y.astype(f32), jnp.zeros((NP - NCTR,), f32)]).reshape(NP, 1)

    kemb = p["key_emb"].astype(f32)
    cemb = p["ctr_emb"].astype(f32)
    vwt = p["val_w"].T.astype(f32)
    vb = p["val_b"].astype(f32).reshape(1, D)
    c2k = _mlp_args(p["msg_c2k"])
    k2c = _mlp_args(p["msg_k2c"])

    def gru_args(g):
        return (g["wih"].T.astype(f32), g["bih"].astype(f32).reshape(1, D3),
                g["whh"].T.astype(f32), g["bhh"].astype(f32).reshape(1, D3))

    gk = gru_args(p["gru_key"])
    gc = gru_args(p["gru_ctr"])
    lnk = (p["ln_key_g"].astype(f32).reshape(1, D), p["ln_key_b"].astype(f32).reshape(1, D))
    lnc = (p["ln_ctr_g"].astype(f32).reshape(1, D), p["ln_ctr_b"].astype(f32).reshape(1, D))

    hkey, crows = _sc_prologue()(kemb, cemb, khp, chp)
    ones128 = jnp.ones((CHUNK, D), f32)
    degk, degc = _sc_deg()(src, dst, ones128, zrows)
    hctr, tab = _tc_prologue(crows, yp, vwt, vb, *c2k)
    agg_fn = _sc_agg()
    for l in range(3):
        aggp = agg_fn(tab, dst, src, zrows)
        hkey, tab2 = _tc_update_tab(aggp, degk, hkey, *gk, *lnk, *k2c)
        aggp2 = agg_fn(tab2, src, dst, zrows)
        if l < 2:
            hctr, tab = _tc_update_tab(aggp2, degc, hctr, *gc, *lnc, *c2k)
        else:
            (hctr,) = _tc_update_notab(aggp2, degc, hctr, *gc, *lnc)
    hkc, hkv, sc_, sv_ = _tc_epilogue(hkey, *_mlp_args(p["proj_key_c"]),
                                      *_mlp_args(p["proj_key_v"]))
    zc, zv = _tc_pool(sc_, sv_, *_mlp_args(p["pool_zc"]), *_mlp_args(p["pool_zv"]))
    f64 = jnp.float64
    return (zc.reshape(D).astype(f64), zv.reshape(D).astype(f64),
            hkc[:NKEY].astype(f64), hkv[:NKEY].astype(f64),
            hctr[:NCTR].astype(f64))


# speedup vs baseline: 1.3961x; 1.0126x over previous
import functools

import jax
import jax.numpy as jnp
from jax import lax
from jax.experimental import pallas as pl
from jax.experimental.pallas import tpu as pltpu
from jax.experimental.pallas import tpu_sc as plsc

D = 128
D3 = 3 * D
NKEY = 10000
NCTR = 10000
NP = 10240
NE = 320000
CHUNK = 128
NTILES = 32
CPT = 79
EP = NTILES * CPT * CHUNK
PAD_IDX = 10000
RPT = NP // NTILES
RPS = NP // 16
NBLK = NP // 128
KEY_BUCKETS = 100000
CTR_BUCKETS = 100000

_PREC = jax.lax.Precision.HIGHEST

_MESH = dict(core_axis_name="c", subcore_axis_name="s", num_cores=2, num_subcores=16)


def _sc_prologue_body(kemb, cemb, kh, ch,
                      hkey_out, crow_out,
                      idx128, idx64, rows128, rows64, sem):
    cid = lax.axis_index("c")
    sid = lax.axis_index("s")
    wid = sid * 2 + cid
    gb = wid * RPT
    for off, cnt, ib, rb in ((0, CHUNK, idx128, rows128),
                             (CHUNK, CHUNK, idx128, rows128),
                             (2 * CHUNK, 64, idx64, rows64)):
        pltpu.sync_copy(kh.at[pl.ds(gb + off, cnt)], ib)
        pltpu.async_copy(kemb.at[ib], rb, sem).wait()
        pltpu.sync_copy(rb, hkey_out.at[pl.ds(gb + off, cnt)])
        pltpu.sync_copy(ch.at[pl.ds(gb + off, cnt)], ib)
        pltpu.async_copy(cemb.at[ib], rb, sem).wait()
        pltpu.sync_copy(rb, crow_out.at[pl.ds(gb + off, cnt)])


@functools.cache
def _sc_prologue():
    return pl.kernel(
        _sc_prologue_body,
        mesh=plsc.VectorSubcoreMesh(**_MESH),
        out_type=[
            jax.ShapeDtypeStruct((NP, D), jnp.float32),
            jax.ShapeDtypeStruct((NP, D), jnp.float32),
        ],
        scratch_types=[
            pltpu.VMEM((CHUNK,), jnp.int32),
            pltpu.VMEM((64,), jnp.int32),
            pltpu.VMEM((CHUNK, D), jnp.float32),
            pltpu.VMEM((64, D), jnp.float32),
            pltpu.SemaphoreType.DMA,
        ],
    )


def _sc_agg_body(table, gidx2, sidx, zrows, out, gslab_v, si_v, rows_v, acc, sem):
    cid = lax.axis_index("c")
    sid = lax.axis_index("s")
    wid = sid * 2 + cid
    pltpu.sync_copy(zrows, acc.at[pl.ds(sid * RPS, RPS)])
    pltpu.sync_copy(gidx2.at[wid], gslab_v)
    plsc.subcore_barrier()

    def body(t, carry):
        gbase = (wid * CPT + t) * CHUNK
        pltpu.sync_copy(sidx.at[pl.ds(gbase, CHUNK)], si_v)
        pltpu.async_copy(table.at[gslab_v.at[t]], rows_v, sem).wait()
        pltpu.sync_copy(rows_v, acc.at[si_v], add=True)
        return carry

    lax.fori_loop(jnp.int32(0), jnp.int32(CPT), body, jnp.int32(0))
    plsc.subcore_barrier()
    pltpu.sync_copy(acc.at[pl.ds(sid * RPS, RPS)],
                    out.at[cid, pl.ds(sid * RPS, RPS)])


@functools.cache
def _sc_agg():
    return pl.kernel(
        _sc_agg_body,
        mesh=plsc.VectorSubcoreMesh(**_MESH),
        out_type=jax.ShapeDtypeStruct((2, NP, D), jnp.float32),
        scratch_types=[
            pltpu.VMEM((CPT, CHUNK), jnp.int32),
            pltpu.VMEM((CHUNK,), jnp.int32),
            pltpu.VMEM((CHUNK, D), jnp.float32),
            pltpu.VMEM_SHARED((NP, D), jnp.float32),
            pltpu.SemaphoreType.DMA,
        ],
    )


def _sc_deg_body(srci, dsti, ones_hbm, zrows, outk, outc, idx_v, ones_v, acc, sem):
    cid = lax.axis_index("c")
    sid = lax.axis_index("s")
    wid = sid * 2 + cid
    slab = pl.ds(sid * RPS, RPS)
    pltpu.sync_copy(zrows, acc.at[slab])
    pltpu.sync_copy(ones_hbm, ones_v)
    plsc.subcore_barrier()

    def body_src(t, carry):
        gbase = (wid * CPT + t) * CHUNK
        pltpu.sync_copy(srci.at[pl.ds(gbase, CHUNK)], idx_v)
        pltpu.sync_copy(ones_v, acc.at[idx_v], add=True)
        return carry

    lax.fori_loop(jnp.int32(0), jnp.int32(CPT), body_src, jnp.int32(0))
    plsc.subcore_barrier()
    pltpu.sync_copy(acc.at[slab], outk.at[cid, slab])
    pltpu.sync_copy(zrows, acc.at[slab])
    plsc.subcore_barrier()

    def body_dst(t, carry):
        gbase = (wid * CPT + t) * CHUNK
        pltpu.sync_copy(dsti.at[pl.ds(gbase, CHUNK)], idx_v)
        pltpu.sync_copy(ones_v, acc.at[idx_v], add=True)
        return carry

    lax.fori_loop(jnp.int32(0), jnp.int32(CPT), body_dst, jnp.int32(0))
    plsc.subcore_barrier()
    pltpu.sync_copy(acc.at[slab], outc.at[cid, slab])


@functools.cache
def _sc_deg():
    return pl.kernel(
        _sc_deg_body,
        mesh=plsc.VectorSubcoreMesh(**_MESH),
        out_type=[
            jax.ShapeDtypeStruct((2, NP, D), jnp.float32),
            jax.ShapeDtypeStruct((2, NP, D), jnp.float32),
        ],
        scratch_types=[
            pltpu.VMEM((CHUNK,), jnp.int32),
            pltpu.VMEM((CHUNK, D), jnp.float32),
            pltpu.VMEM_SHARED((NP, D), jnp.float32),
            pltpu.SemaphoreType.DMA,
        ],
    )


def _full(shape):
    return pl.BlockSpec(shape, lambda i: tuple(jnp.int32(0) for _ in shape))


def _rows(width):
    return pl.BlockSpec((128, width), lambda i: (i, jnp.int32(0)))


def _mlp_blk(x, w1t, b1, w2t, b2):
    h = jnp.maximum(jnp.dot(x, w1t, precision=_PREC) + b1, 0.0)
    return jnp.dot(h, w2t, precision=_PREC) + b2


def _tc_prologue_body(crow_ref, y_ref, vwt_ref, vb_ref,
                      w1t_ref, b1_ref, w2t_ref, b2_ref,
                      hctr_ref, tab_ref):
    y = y_ref[...]
    l1p = jnp.log1p(jnp.maximum(y, 0.0))
    pre = y * vwt_ref[0:1, :] + l1p * vwt_ref[1:2, :] + vb_ref[...]
    hc = crow_ref[...] + jnp.maximum(pre, 0.0)
    hctr_ref[...] = hc
    tab_ref[...] = _mlp_blk(hc, w1t_ref[...], b1_ref[...], w2t_ref[...], b2_ref[...])


_tc_prologue = pl.pallas_call(
    _tc_prologue_body,
    grid=(NBLK,),
    in_specs=[_rows(D), _rows(1), _full((2, D)), _full((1, D)),
              _full((D, D)), _full((1, D)), _full((D, D)), _full((1, D))],
    out_specs=[_rows(D), _rows(D)],
    out_shape=[jax.ShapeDtypeStruct((NP, D), jnp.float32),
               jax.ShapeDtypeStruct((NP, D), jnp.float32)],
)


def _mk_update(with_table):
    def body(p_ref, dp_ref, h_ref, wiht_ref, bih_ref, whht_ref, bhh_ref,
             g_ref, b_ref, *rest):
        x = (p_ref[0] + p_ref[1]) / jnp.maximum(dp_ref[0, :, 0:1] + dp_ref[1, :, 0:1], 1.0)
        h = h_ref[...]
        gi = jnp.dot(x, wiht_ref[...], precision=_PREC) + bih_ref[...]
        gh = jnp.dot(h, whht_ref[...], precision=_PREC) + bhh_ref[...]
        r = jax.nn.sigmoid(gi[:, :D] + gh[:, :D])
        z = jax.nn.sigmoid(gi[:, D:2 * D] + gh[:, D:2 * D])
        n = jnp.tanh(gi[:, 2 * D:] + r * gh[:, 2 * D:])
        hn = (1.0 - z) * n + z * h
        mu = jnp.mean(hn, axis=1, keepdims=True)
        var = jnp.mean((hn - mu) ** 2, axis=1, keepdims=True)
        ln = (hn - mu) / jnp.sqrt(var + 1e-5) * g_ref[...] + b_ref[...]
        if with_table:
            w1t_ref, b1_ref, w2t_ref, b2_ref, hout_ref, tab_ref = rest
            hout_ref[...] = ln
            tab_ref[...] = _mlp_blk(ln, w1t_ref[...], b1_ref[...], w2t_ref[...], b2_ref[...])
        else:
            (hout_ref,) = rest
            hout_ref[...] = ln

    in_specs = [
        pl.BlockSpec((2, 128, D), lambda i: (jnp.int32(0), i, jnp.int32(0))),
        pl.BlockSpec((2, 128, D), lambda i: (jnp.int32(0), i, jnp.int32(0))),
        _rows(D),
        _full((D, D3)), _full((1, D3)), _full((D, D3)), _full((1, D3)),
        _full((1, D)), _full((1, D)),
    ]
    out_shape = [jax.ShapeDtypeStruct((NP, D), jnp.float32)]
    out_specs = [_rows(D)]
    if with_table:
        in_specs += [_full((D, D)), _full((1, D)), _full((D, D)), _full((1, D))]
        out_shape += [jax.ShapeDtypeStruct((NP, D), jnp.float32)]
        out_specs += [_rows(D)]
    return pl.pallas_call(
        body, grid=(NBLK,), in_specs=in_specs,
        out_specs=out_specs, out_shape=out_shape,
    )


_tc_update_tab = _mk_update(True)
_tc_update_notab = _mk_update(False)


def _tc_epilogue_body(h_ref, cw1t, cb1, cw2t, cb2, vw1t, vb1, vw2t, vb2,
                      hkc_ref, hkv_ref, sc_ref, sv_ref):
    i = pl.program_id(0)
    h = h_ref[...]
    hc = _mlp_blk(h, cw1t[...], cb1[...], cw2t[...], cb2[...])
    hv = _mlp_blk(h, vw1t[...], vb1[...], vw2t[...], vb2[...])
    hkc_ref[...] = hc
    hkv_ref[...] = hv
    rows = i * 128 + lax.broadcasted_iota(jnp.int32, (128, 1), 0)
    mask = rows < NKEY

    @pl.when(i == 0)
    def _():
        sc_ref[...] = jnp.zeros((1, D), jnp.float32)
        sv_ref[...] = jnp.zeros((1, D), jnp.float32)

    sc_ref[...] += jnp.sum(jnp.where(mask, hc, 0.0), axis=0, keepdims=True)
    sv_ref[...] += jnp.sum(jnp.where(mask, hv, 0.0), axis=0, keepdims=True)


_tc_epilogue = pl.pallas_call(
    _tc_epilogue_body,
    grid=(NBLK,),
    in_specs=[_rows(D)] + [_full((D, D)), _full((1, D)), _full((D, D)), _full((1, D))] * 2,
    out_specs=[_rows(D), _rows(D),
               pl.BlockSpec((1, D), lambda i: (jnp.int32(0), jnp.int32(0))),
               pl.BlockSpec((1, D), lambda i: (jnp.int32(0), jnp.int32(0)))],
    out_shape=[jax.ShapeDtypeStruct((NP, D), jnp.float32),
               jax.ShapeDtypeStruct((NP, D), jnp.float32),
               jax.ShapeDtypeStruct((1, D), jnp.float32),
               jax.ShapeDtypeStruct((1, D), jnp.float32)],
)


def _tc_pool_body(sc_ref, sv_ref, cw1t, cb1, cw2t, cb2, vw1t, vb1, vw2t, vb2,
                  zc_ref, zv_ref):
    zc_ref[...] = _mlp_blk(sc_ref[...] * (1.0 / NKEY), cw1t[...], cb1[...], cw2t[...], cb2[...])
    zv_ref[...] = _mlp_blk(sv_ref[...] * (1.0 / NKEY), vw1t[...], vb1[...], vw2t[...], vb2[...])


_tc_pool = pl.pallas_call(
    _tc_pool_body,
    out_shape=[jax.ShapeDtypeStruct((1, D), jnp.float32),
               jax.ShapeDtypeStruct((1, D), jnp.float32)],
)


def _mlp_args(p):
    return (p["w1"].T.astype(jnp.float32), p["b1"].astype(jnp.float32).reshape(1, D),
            p["w2"].T.astype(jnp.float32), p["b2"].astype(jnp.float32).reshape(1, D))


def kernel(y, edge_index, key_ids, counter_ids_global, params):
    p = params
    f32 = jnp.float32
    mult = jnp.uint64(2654435761)
    kh = ((key_ids.astype(jnp.uint64) * mult) % jnp.uint64(KEY_BUCKETS)).astype(jnp.int32)
    ch = ((counter_ids_global.astype(jnp.uint64) * mult) % jnp.uint64(CTR_BUCKETS)).astype(jnp.int32)
    khp = jnp.concatenate([kh, jnp.zeros((NP - NKEY,), jnp.int32)])
    chp = jnp.concatenate([ch, jnp.zeros((NP - NCTR,), jnp.int32)])
    pad = jnp.full((EP - NE,), PAD_IDX, jnp.int32)
    src = jnp.concatenate([edge_index[0].astype(jnp.int32), pad])
    dst = jnp.concatenate([edge_index[1].astype(jnp.int32), pad])
    src3 = src.reshape(NTILES, CPT, CHUNK)
    dst3 = dst.reshape(NTILES, CPT, CHUNK)
    zrows = jnp.zeros((RPS, D), f32)
    yp = jnp.concatenate([y.astype(f32), jnp.zeros((NP - NCTR,), f32)]).reshape(NP, 1)

    kemb = p["key_emb"].astype(f32)
    cemb = p["ctr_emb"].astype(f32)
    vwt = p["val_w"].T.astype(f32)
    vb = p["val_b"].astype(f32).reshape(1, D)
    c2k = _mlp_args(p["msg_c2k"])
    k2c = _mlp_args(p["msg_k2c"])

    def gru_args(g):
        return (g["wih"].T.astype(f32), g["bih"].astype(f32).reshape(1, D3),
                g["whh"].T.astype(f32), g["bhh"].astype(f32).reshape(1, D3))

    gk = gru_args(p["gru_key"])
    gc = gru_args(p["gru_ctr"])
    lnk = (p["ln_key_g"].astype(f32).reshape(1, D), p["ln_key_b"].astype(f32).reshape(1, D))
    lnc = (p["ln_ctr_g"].astype(f32).reshape(1, D), p["ln_ctr_b"].astype(f32).reshape(1, D))

    hkey, crows = _sc_prologue()(kemb, cemb, khp, chp)
    ones128 = jnp.ones((CHUNK, D), f32)
    degk, degc = _sc_deg()(src, dst, ones128, zrows)
    hctr, tab = _tc_prologue(crows, yp, vwt, vb, *c2k)
    agg_fn = _sc_agg()
    for l in range(3):
        aggp = agg_fn(tab, dst3, src, zrows)
        hkey, tab2 = _tc_update_tab(aggp, degk, hkey, *gk, *lnk, *k2c)
        aggp2 = agg_fn(tab2, src3, dst, zrows)
        if l < 2:
            hctr, tab = _tc_update_tab(aggp2, degc, hctr, *gc, *lnc, *c2k)
        else:
            (hctr,) = _tc_update_notab(aggp2, degc, hctr, *gc, *lnc)
    hkc, hkv, sc_, sv_ = _tc_epilogue(hkey, *_mlp_args(p["proj_key_c"]),
                                      *_mlp_args(p["proj_key_v"]))
    zc, zv = _tc_pool(sc_, sv_, *_mlp_args(p["pool_zc"]), *_mlp_args(p["pool_zv"]))
    f64 = jnp.float64
    return (zc.reshape(D).astype(f64), zv.reshape(D).astype(f64),
            hkc[:NKEY].astype(f64), hkv[:NKEY].astype(f64),
            hctr[:NCTR].astype(f64))
